# baseline (speedup 1.0000x reference)
import functools
import math

import jax
import jax.numpy as jnp
from jax import lax
from jax.experimental import pallas as pl
from jax.experimental.pallas import tpu as pltpu
from jax.experimental.pallas import tpu_sc as plsc

D = 512
L_SEQ = 512
SCALE = math.sqrt(float(D))
NBUF = 2

try:
    _info = plsc.get_sparse_core_info()
    NC, NS, LANES = _info.num_cores, _info.num_subcores, _info.num_lanes
except Exception:
    NC, NS, LANES = 2, 16, 16
NW = NC * NS
NLANE = D // LANES


def _make_lookup(total_rows: int, chunk_rows: int):
    rows_w = total_rows // NW
    nchunk = rows_w // chunk_rows
    ngroup = nchunk // NBUF
    kblk = L_SEQ // (NW // 2)
    assert kblk == chunk_rows

    mesh = plsc.VectorSubcoreMesh(core_axis_name="c", subcore_axis_name="s")

    @functools.partial(
        pl.kernel,
        mesh=mesh,
        out_type=jax.ShapeDtypeStruct((total_rows, D), jnp.float32),
        scratch_types=[
            pltpu.VMEM((nchunk, chunk_rows), jnp.int32),
            pltpu.VMEM((NBUF, chunk_rows, D // 2), jnp.int32),
            pltpu.VMEM((NBUF, chunk_rows, D), jnp.float32),
            pltpu.VMEM((chunk_rows, D), jnp.float32),
            pltpu.SemaphoreType.DMA,
            pltpu.SemaphoreType.DMA,
            pltpu.SemaphoreType.DMA,
            pltpu.SemaphoreType.DMA,
        ],
    )
    def lookup(idx_hbm, table_hbm, pos_hbm, out_hbm,
               idx_v, gbuf, obuf, pos_l, g0, g1, o0, o1):
        cid = lax.axis_index("c")
        sid = lax.axis_index("s")
        wid = sid * NC + cid
        s_stack = wid // (NW // 2)
        kpos = wid % (NW // 2)
        gsem = (g0, g1)
        osem = (o0, o1)

        pltpu.sync_copy(idx_hbm.at[pl.ds(wid * nchunk, nchunk)], idx_v)
        pltpu.sync_copy(
            pos_hbm.at[pl.ds(s_stack * L_SEQ + kpos * kblk, kblk)], pos_l)

        out_base = s_stack * (total_rows // 2) + kpos * kblk

        def issue_gather(c, b):
            pltpu.async_copy(table_hbm.at[idx_v.at[c]], gbuf.at[b], gsem[b])

        def wait_gather(c, b):
            pltpu.make_async_copy(table_hbm.at[idx_v.at[c]],
                                  gbuf.at[b], gsem[b]).wait()

        def issue_out(c, b):
            pltpu.async_copy(obuf.at[b],
                             out_hbm.at[pl.ds(out_base + c * L_SEQ,
                                              chunk_rows)], osem[b])

        def wait_out(b):
            pltpu.make_async_copy(obuf.at[b],
                                  out_hbm.at[pl.ds(out_base, chunk_rows)],
                                  osem[b]).wait()

        def combine(b):
            gb = gbuf.at[b]
            ob = obuf.at[b]

            def row_body(r, carry):
                for j in range(NLANE // 2):
                    w = gb[r, pl.ds(j * LANES, LANES)]
                    a = lax.bitcast_convert_type(
                        lax.shift_left(w, 16), jnp.float32)
                    b2 = lax.bitcast_convert_type(
                        lax.bitwise_and(w, jnp.int32(-65536)), jnp.float32)
                    s0 = pl.ds(j * 32, LANES)
                    s1 = pl.ds(j * 32 + LANES, LANES)
                    ob[r, s0] = a + pos_l[r, s0]
                    ob[r, s1] = b2 + pos_l[r, s1]
                return carry

            lax.fori_loop(0, chunk_rows, row_body, 0)

        issue_gather(0, 0)

        def group_body(g, carry):
            for b in range(NBUF):
                c = g * NBUF + b
                nb = (b + 1) % NBUF

                @pl.when(c + 1 < nchunk)
                def _():
                    issue_gather(c + 1, nb)

                wait_gather(c, b)

                @pl.when(c >= NBUF)
                def _():
                    wait_out(b)

                combine(b)
                issue_out(c, b)
            return carry

        lax.fori_loop(0, ngroup, group_body, 0)
        for b in range(NBUF):
            wait_out(b)

    return lookup


def kernel(src, tgt, emb_table, pos_src_table, pos_tgt_table):
    B, L = src.shape
    _, LP = tgt.shape
    total_rows = B * L + B * LP
    chunk_rows = 32
    kw = NW // 2
    idx_all = jnp.stack([src, tgt])
    idx_perm = idx_all.reshape(2, B, kw, chunk_rows).transpose(0, 2, 1, 3)
    idx_2d = idx_perm.reshape(2 * kw * B, chunk_rows)
    pos_cat = jnp.concatenate([pos_src_table, pos_tgt_table], axis=0)
    tb = (emb_table * SCALE).astype(jnp.bfloat16)
    V = tb.shape[0]
    t4 = tb.reshape(V, NLANE // 2, 2, LANES)
    tab_il = t4.transpose(0, 1, 3, 2).reshape(V, D // 2, 2)
    tab_i32 = jax.lax.bitcast_convert_type(tab_il, jnp.int32)
    flat = _make_lookup(total_rows, chunk_rows)(idx_2d, tab_i32, pos_cat)
    return flat.reshape(2, B, L, D)


# speedup vs baseline: 2.0837x; 2.0837x over previous
import functools
import math

import jax
import jax.numpy as jnp
from jax import lax
from jax.experimental import pallas as pl
from jax.experimental.pallas import tpu as pltpu
from jax.experimental.pallas import tpu_sc as plsc

D = 512
L_SEQ = 512
SCALE = math.sqrt(float(D))
NBUF = 2

try:
    _info = plsc.get_sparse_core_info()
    NC, NS, LANES = _info.num_cores, _info.num_subcores, _info.num_lanes
except Exception:
    NC, NS, LANES = 2, 16, 16
NW = NC * NS
NLANE = D // LANES


def _make_lookup(total_rows: int, chunk_rows: int):
    rows_w = total_rows // NW
    nchunk = rows_w // chunk_rows
    ngroup = nchunk // NBUF
    kblk = L_SEQ // (NW // 2)
    assert kblk == chunk_rows

    mesh = plsc.VectorSubcoreMesh(core_axis_name="c", subcore_axis_name="s")

    @functools.partial(
        pl.kernel,
        mesh=mesh,
        out_type=jax.ShapeDtypeStruct((total_rows, D), jnp.float32),
        scratch_types=[
            pltpu.VMEM((nchunk, chunk_rows), jnp.int32),
            pltpu.VMEM((NBUF, chunk_rows, D // 2), jnp.int32),
            pltpu.VMEM((NBUF, chunk_rows, D), jnp.float32),
            pltpu.VMEM((chunk_rows, D), jnp.float32),
            pltpu.SemaphoreType.DMA,
            pltpu.SemaphoreType.DMA,
            pltpu.SemaphoreType.DMA,
            pltpu.SemaphoreType.DMA,
        ],
    )
    def lookup(idx_hbm, table_hbm, pos_hbm, out_hbm,
               idx_v, gbuf, obuf, pos_l, g0, g1, o0, o1):
        cid = lax.axis_index("c")
        sid = lax.axis_index("s")
        wid = sid * NC + cid
        s_stack = wid // (NW // 2)
        kpos = wid % (NW // 2)
        gsem = (g0, g1)
        osem = (o0, o1)

        pltpu.sync_copy(idx_hbm.at[pl.ds(wid * nchunk, nchunk)], idx_v)
        pltpu.sync_copy(
            pos_hbm.at[pl.ds(s_stack * L_SEQ + kpos * kblk, kblk)], pos_l)

        out_base = s_stack * (total_rows // 2) + kpos * kblk

        def issue_gather(c, b):
            pltpu.async_copy(table_hbm.at[idx_v.at[c]], gbuf.at[b], gsem[b])

        def wait_gather(c, b):
            pltpu.make_async_copy(table_hbm.at[idx_v.at[c]],
                                  gbuf.at[b], gsem[b]).wait()

        def issue_out(c, b):
            pltpu.async_copy(obuf.at[b],
                             out_hbm.at[pl.ds(out_base + c * L_SEQ,
                                              chunk_rows)], osem[b])

        def wait_out(b):
            pltpu.make_async_copy(obuf.at[b],
                                  out_hbm.at[pl.ds(out_base, chunk_rows)],
                                  osem[b]).wait()

        def combine(b):
            gb = gbuf.at[b]
            ob = obuf.at[b]

            @plsc.parallel_loop(0, chunk_rows, unroll=4)
            def _(r):
                for j in range(NLANE // 2):
                    w = gb[r, pl.ds(j * LANES, LANES)]
                    a = lax.bitcast_convert_type(
                        lax.shift_left(w, 16), jnp.float32)
                    b2 = lax.bitcast_convert_type(
                        lax.bitwise_and(w, jnp.int32(-65536)), jnp.float32)
                    s0 = pl.ds(j * 32, LANES)
                    s1 = pl.ds(j * 32 + LANES, LANES)
                    ob[r, s0] = a + pos_l[r, s0]
                    ob[r, s1] = b2 + pos_l[r, s1]

        issue_gather(0, 0)

        def group_body(g, carry):
            for b in range(NBUF):
                c = g * NBUF + b
                nb = (b + 1) % NBUF

                @pl.when(c + 1 < nchunk)
                def _():
                    issue_gather(c + 1, nb)

                wait_gather(c, b)

                @pl.when(c >= NBUF)
                def _():
                    wait_out(b)

                combine(b)
                issue_out(c, b)
            return carry

        lax.fori_loop(0, ngroup, group_body, 0)
        for b in range(NBUF):
            wait_out(b)

    return lookup


def kernel(src, tgt, emb_table, pos_src_table, pos_tgt_table):
    B, L = src.shape
    _, LP = tgt.shape
    total_rows = B * L + B * LP
    chunk_rows = 32
    kw = NW // 2
    idx_all = jnp.stack([src, tgt])
    idx_perm = idx_all.reshape(2, B, kw, chunk_rows).transpose(0, 2, 1, 3)
    idx_2d = idx_perm.reshape(2 * kw * B, chunk_rows)
    pos_cat = jnp.concatenate([pos_src_table, pos_tgt_table], axis=0)
    tb = (emb_table * SCALE).astype(jnp.bfloat16)
    V = tb.shape[0]
    t4 = tb.reshape(V, NLANE // 2, 2, LANES)
    tab_il = t4.transpose(0, 1, 3, 2).reshape(V, D // 2, 2)
    tab_i32 = jax.lax.bitcast_convert_type(tab_il, jnp.int32)
    flat = _make_lookup(total_rows, chunk_rows)(idx_2d, tab_i32, pos_cat)
    return flat.reshape(2, B, L, D)
